# baseline (speedup 1.0000x reference)
import functools

import jax
import jax.numpy as jnp
from jax.experimental import pallas as pl
from jax.experimental.pallas import tpu as pltpu
from jax.experimental.pallas import tpu_sc as plsc

N_NODES = 10000
N_EDGES = 320000
D_NODE = 128
D_EDGE = 16
D_OUT = 128
D_HID = 256

NC = 2
NS = 16
NW = NC * NS

E_PER_W = N_EDGES // NW
CHUNK = 80
N_CHUNKS = E_PER_W // CHUNK

N_PAD = 10240
ROWS_PER_SUB = N_PAD // NS
ZROWS = 128

_vector_mesh = plsc.VectorSubcoreMesh(core_axis_name="c", subcore_axis_name="s")


GRP = 5
K_SPLIT = 5
E_SPLIT = N_EDGES // K_SPLIT
CE_PER_W = E_SPLIT // NW
CN_CHUNKS = CE_PER_W // CHUNK
CN_GRP = CN_CHUNKS // GRP


D_PK = D_NODE // 2


def _gather_body(nodes_hbm, senders_hbm, receivers_hbm, s_out, r_out,
                 idx_all, *rest):
    bufs = (rest[0:GRP], rest[GRP:2 * GRP])
    gsem = rest[2 * GRP:2 * GRP + 2]
    wsem = rest[2 * GRP + 2:2 * GRP + 4]
    wid = jax.lax.axis_index("s") * NC + jax.lax.axis_index("c")
    base = wid * CE_PER_W

    def phase(src_hbm, out_hbm):
        pltpu.sync_copy(src_hbm.at[pl.ds(base, CE_PER_W)], idx_all)

        def fire_g(g, h):
            for b in range(GRP):
                idx = idx_all.at[pl.ds((g * GRP + b) * CHUNK, CHUNK)]
                pltpu.async_copy(nodes_hbm.at[idx], bufs[h][b], gsem[h])

        def drain_g(h):
            for b in range(GRP):
                pltpu.make_async_copy(
                    nodes_hbm.at[pl.ds(0, CHUNK)], bufs[h][b], gsem[h]).wait()

        def fire_w(g, h):
            for b in range(GRP):
                dst = out_hbm.at[pl.ds(base + (g * GRP + b) * CHUNK, CHUNK)]
                pltpu.async_copy(bufs[h][b], dst, wsem[h])

        def drain_w(h):
            for b in range(GRP):
                pltpu.make_async_copy(
                    bufs[h][b], out_hbm.at[pl.ds(base, CHUNK)], wsem[h]).wait()

        fire_g(0, 0)
        fire_g(1, 1); drain_g(0); fire_w(0, 0)

        @pl.loop(0, (CN_GRP - 3) // 2)
        def _(u):
            t1 = 2 * u + 1
            drain_w(0); fire_g(t1 + 1, 0); drain_g(1); fire_w(t1, 1)
            t2 = 2 * u + 2
            drain_w(1); fire_g(t2 + 1, 1); drain_g(0); fire_w(t2, 0)

        drain_w(0); fire_g(CN_GRP - 1, 0); drain_g(1); fire_w(CN_GRP - 2, 1)
        drain_w(1); drain_g(0); fire_w(CN_GRP - 1, 0)
        drain_w(0)

    phase(senders_hbm, s_out)
    phase(receivers_hbm, r_out)


@jax.jit
def _sc_gather(nodes_pk, senders, receivers):
    feat = jax.ShapeDtypeStruct((E_SPLIT, D_PK), jnp.int32)
    k = pl.kernel(
        _gather_body,
        out_type=(feat, feat),
        mesh=_vector_mesh,
        compiler_params=pltpu.CompilerParams(use_tc_tiling_on_sc=False),
        scratch_types=[
            pltpu.VMEM((CE_PER_W,), jnp.int32),
            *[pltpu.VMEM((CHUNK, D_PK), jnp.int32) for _ in range(2 * GRP)],
            pltpu.SemaphoreType.DMA,
            pltpu.SemaphoreType.DMA,
            pltpu.SemaphoreType.DMA,
            pltpu.SemaphoreType.DMA,
        ],
    )
    return k(nodes_pk, senders, receivers)


SCAT_RING = 3


def _scatter_body(m0, m1, m2, m3, m4, recv4_hbm, p0_out, p1_out,
                  acc_sh, idx2_v, *rest):
    msg_arrays = (m0, m1, m2, m3, m4)
    bufs = rest[0:SCAT_RING]
    lsem = rest[SCAT_RING:2 * SCAT_RING]
    cid = jax.lax.axis_index("c")
    sid = jax.lax.axis_index("s")
    wid = sid * NC + cid
    base = wid * CE_PER_W

    zb = bufs[0]

    @pl.loop(0, CHUNK)
    def _(r):
        @pl.loop(0, D_NODE // 16)
        def _(cc):
            zb[r, pl.ds(cc * 16, 16)] = jnp.zeros((16,), jnp.float32)

    @pl.loop(0, ROWS_PER_SUB // CHUNK)
    def _(t):
        pltpu.sync_copy(zb, acc_sh.at[pl.ds(sid * ROWS_PER_SUB + t * CHUNK, CHUNK)])

    plsc.subcore_barrier()

    for k in range(K_SPLIT):
        mk = msg_arrays[k]
        pltpu.sync_copy(recv4_hbm.at[k, wid], idx2_v)

        def fire_l(j, b, mk=mk):
            src = mk.at[pl.ds(base + j * CHUNK, CHUNK)]
            pltpu.async_copy(src, bufs[b], lsem[b])

        def drain_l(b, mk=mk):
            pltpu.make_async_copy(
                mk.at[pl.ds(0, CHUNK)], bufs[b], lsem[b]).wait()

        def scat(j, b):
            pltpu.sync_copy(bufs[b], acc_sh.at[idx2_v.at[j]], add=True)

        for b in range(SCAT_RING):
            fire_l(b, b)

        @pl.loop(0, (CN_CHUNKS - 7) // SCAT_RING)
        def _(g):
            for b in range(SCAT_RING):
                j = g * SCAT_RING + b
                drain_l(b)
                scat(j, b)
                fire_l(j + SCAT_RING, b)

        for j in range(CN_CHUNKS - 7, CN_CHUNKS):
            b = j % SCAT_RING
            drain_l(b)
            scat(j, b)
            if j + SCAT_RING < CN_CHUNKS:
                fire_l(j + SCAT_RING, b)

    plsc.subcore_barrier()

    rb = sid * ROWS_PER_SUB

    @pl.when(cid == 0)
    def _():
        pltpu.sync_copy(acc_sh.at[pl.ds(rb, ROWS_PER_SUB)], p0_out.at[pl.ds(rb, ROWS_PER_SUB)])

    @pl.when(cid == 1)
    def _():
        pltpu.sync_copy(acc_sh.at[pl.ds(rb, ROWS_PER_SUB)], p1_out.at[pl.ds(rb, ROWS_PER_SUB)])


@jax.jit
def _sc_scatter(m0, m1, m2, m3, m4, receivers):
    part = jax.ShapeDtypeStruct((N_PAD, D_NODE), jnp.float32)
    recv4 = receivers.reshape(K_SPLIT, NW, CN_CHUNKS, CHUNK)
    k = pl.kernel(
        _scatter_body,
        out_type=(part, part),
        mesh=_vector_mesh,
        scratch_types=[
            pltpu.VMEM_SHARED((N_PAD, D_NODE), jnp.float32),
            pltpu.VMEM((CN_CHUNKS, CHUNK), jnp.int32),
            *[pltpu.VMEM((CHUNK, D_NODE), jnp.float32) for _ in range(SCAT_RING)],
            *[pltpu.SemaphoreType.DMA for _ in range(SCAT_RING)],
        ],
    )
    return k(m0, m1, m2, m3, m4, recv4)


E_BLK = 2000


D_EEXT = 32


def _mlp_body(sr_ref, e_ref, w1sr_ref, w1e_ref, w2_ref, b2_ref, o_ref):
    x = sr_ref[...]
    h = jnp.dot(x, w1sr_ref[...], preferred_element_type=jnp.float32)
    h += jnp.dot(e_ref[...], w1e_ref[...], preferred_element_type=jnp.float32)
    h = jnp.maximum(h, 0.0).astype(jnp.bfloat16)
    o_ref[...] = jnp.dot(h, w2_ref[...], preferred_element_type=jnp.float32) + b2_ref[...]


def _mlp_chunk(k, sr_feat, e_ext, w1sr, w1e_ext, w2, b2r):
    grid = (E_SPLIT // E_BLK,)
    blocks_per_split = E_SPLIT // E_BLK
    in_specs = [
        pl.BlockSpec((E_BLK, 2 * D_NODE), lambda i: (i, 0)),
        pl.BlockSpec((E_BLK, D_EEXT), lambda i: (i + k * blocks_per_split, 0)),
        pl.BlockSpec((2 * D_NODE, D_HID), lambda i: (0, 0)),
        pl.BlockSpec((D_EEXT, D_HID), lambda i: (0, 0)),
        pl.BlockSpec((D_HID, D_OUT), lambda i: (0, 0)),
        pl.BlockSpec((1, D_OUT), lambda i: (0, 0)),
    ]
    out_spec = pl.BlockSpec((E_BLK, D_OUT), lambda i: (i, 0))
    out_shape = jax.ShapeDtypeStruct((E_SPLIT, D_OUT), jnp.float32)
    return pl.pallas_call(
        _mlp_body, grid=grid, in_specs=in_specs,
        out_specs=out_spec, out_shape=out_shape,
    )(sr_feat, e_ext, w1sr, w1e_ext, w2, b2r)


N_BLK = 2000


def _update_body(p0_ref, p1_ref, n_ref, wn_ref, bn_ref, o_ref):
    proj = jnp.dot(n_ref[...].astype(jnp.bfloat16), wn_ref[...],
                   preferred_element_type=jnp.float32)
    o_ref[...] = p0_ref[...] + p1_ref[...] + proj + bn_ref[...]


def _update(p0, p1, nodes, Wn, bn):
    wn = Wn.astype(jnp.bfloat16)
    bnr = bn.reshape(1, D_OUT)
    return pl.pallas_call(
        _update_body,
        grid=(N_NODES // N_BLK,),
        in_specs=[
            pl.BlockSpec((N_BLK, D_OUT), lambda i: (i, 0)),
            pl.BlockSpec((N_BLK, D_OUT), lambda i: (i, 0)),
            pl.BlockSpec((N_BLK, D_NODE), lambda i: (i, 0)),
            pl.BlockSpec((D_NODE, D_OUT), lambda i: (0, 0)),
            pl.BlockSpec((1, D_OUT), lambda i: (0, 0)),
        ],
        out_specs=pl.BlockSpec((N_BLK, D_OUT), lambda i: (i, 0)),
        out_shape=jax.ShapeDtypeStruct((N_NODES, D_OUT), jnp.float32),
    )(p0, p1, nodes, wn, bnr)


def kernel(nodes, edges, senders, receivers, W1, b1, W2, b2, Wn, bn):
    w1sr = W1[:2 * D_NODE].astype(jnp.bfloat16)
    w1e_ext = jnp.concatenate(
        [W1[2 * D_NODE:], b1.reshape(1, D_HID),
         jnp.zeros((D_EEXT - D_EDGE - 1, D_HID), W1.dtype)],
        axis=0).astype(jnp.bfloat16)
    e_ext = jnp.concatenate(
        [edges, jnp.ones((N_EDGES, 1), edges.dtype),
         jnp.zeros((N_EDGES, D_EEXT - D_EDGE - 1), edges.dtype)],
        axis=1).astype(jnp.bfloat16)
    w2 = W2.astype(jnp.bfloat16)
    b2r = b2.reshape(1, D_OUT)
    nodes_pk = jax.lax.bitcast_convert_type(
        nodes.astype(jnp.bfloat16).reshape(N_NODES, D_PK, 2), jnp.int32)

    def _unpack(x_pk):
        x = jax.lax.bitcast_convert_type(x_pk, jnp.bfloat16)
        return x.reshape(E_SPLIT, D_NODE)

    msgs = []
    for k in range(K_SPLIT):
        sl = slice(k * E_SPLIT, (k + 1) * E_SPLIT)
        s_pk, r_pk = _sc_gather(nodes_pk, senders[sl], receivers[sl])
        sr_feat = jnp.concatenate([_unpack(s_pk), _unpack(r_pk)], axis=1)
        msgs.append(_mlp_chunk(k, sr_feat, e_ext, w1sr, w1e_ext, w2, b2r))
    p0, p1 = _sc_scatter(*msgs, receivers)
    return _update(p0, p1, nodes, Wn, bn)


# speedup vs baseline: 3.7933x; 3.7933x over previous
import functools

import jax
import jax.numpy as jnp
from jax.experimental import pallas as pl
from jax.experimental.pallas import tpu as pltpu
from jax.experimental.pallas import tpu_sc as plsc

N_NODES = 10000
N_EDGES = 320000
D_NODE = 128
D_EDGE = 16
D_OUT = 128
D_HID = 256

NC = 2
NS = 16
NW = NC * NS

E_PER_W = N_EDGES // NW
CHUNK = 80
N_CHUNKS = E_PER_W // CHUNK

N_PAD = 10240
ROWS_PER_SUB = N_PAD // NS
ZROWS = 128

_vector_mesh = plsc.VectorSubcoreMesh(core_axis_name="c", subcore_axis_name="s")


GRP = 5
K_SPLIT = 5
E_SPLIT = N_EDGES // K_SPLIT
CE_PER_W = E_SPLIT // NW
CN_CHUNKS = CE_PER_W // CHUNK
CN_GRP = CN_CHUNKS // GRP


def _gather_body(nodes_hbm, senders_hbm, receivers_hbm, sr_out,
                 idx_all, *rest):
    bufs = (rest[0:GRP], rest[GRP:2 * GRP])
    gsem = rest[2 * GRP:2 * GRP + 2]
    wsem = rest[2 * GRP + 2:2 * GRP + 4]
    wid = jax.lax.axis_index("s") * NC + jax.lax.axis_index("c")
    base = wid * CE_PER_W

    def phase(src_hbm, col):
        pltpu.sync_copy(src_hbm.at[pl.ds(base, CE_PER_W)], idx_all)

        def fire_g(g, h):
            for b in range(GRP):
                idx = idx_all.at[pl.ds((g * GRP + b) * CHUNK, CHUNK)]
                pltpu.async_copy(nodes_hbm.at[idx], bufs[h][b], gsem[h])

        def drain_g(h):
            for b in range(GRP):
                pltpu.make_async_copy(
                    nodes_hbm.at[pl.ds(0, CHUNK)], bufs[h][b], gsem[h]).wait()

        def fire_w(g, h):
            for b in range(GRP):
                dst = sr_out.at[pl.ds(base + (g * GRP + b) * CHUNK, CHUNK),
                                pl.ds(col, D_NODE)]
                pltpu.async_copy(bufs[h][b], dst, wsem[h])

        def drain_w(h):
            for b in range(GRP):
                pltpu.make_async_copy(
                    bufs[h][b],
                    sr_out.at[pl.ds(base, CHUNK), pl.ds(col, D_NODE)],
                    wsem[h]).wait()

        fire_g(0, 0)
        fire_g(1, 1); drain_g(0); fire_w(0, 0)

        @pl.loop(0, (CN_GRP - 3) // 2)
        def _(u):
            t1 = 2 * u + 1
            drain_w(0); fire_g(t1 + 1, 0); drain_g(1); fire_w(t1, 1)
            t2 = 2 * u + 2
            drain_w(1); fire_g(t2 + 1, 1); drain_g(0); fire_w(t2, 0)

        drain_w(0); fire_g(CN_GRP - 1, 0); drain_g(1); fire_w(CN_GRP - 2, 1)
        drain_w(1); drain_g(0); fire_w(CN_GRP - 1, 0)
        drain_w(0)

    phase(senders_hbm, 0)
    phase(receivers_hbm, D_NODE)


@jax.jit
def _sc_gather(nodes, senders, receivers):
    feat = jax.ShapeDtypeStruct((E_SPLIT, 2 * D_NODE), jnp.float32)
    k = pl.kernel(
        _gather_body,
        out_type=feat,
        mesh=_vector_mesh,
        scratch_types=[
            pltpu.VMEM((CE_PER_W,), jnp.int32),
            *[pltpu.VMEM((CHUNK, D_NODE), jnp.float32) for _ in range(2 * GRP)],
            pltpu.SemaphoreType.DMA,
            pltpu.SemaphoreType.DMA,
            pltpu.SemaphoreType.DMA,
            pltpu.SemaphoreType.DMA,
        ],
    )
    return k(nodes, senders, receivers)


SCAT_RING = 3


def _scatter_body(m0, m1, m2, m3, m4, recv4_hbm, p0_out, p1_out,
                  acc_sh, idx2_v, *rest):
    msg_arrays = (m0, m1, m2, m3, m4)
    bufs = rest[0:SCAT_RING]
    lsem = rest[SCAT_RING:2 * SCAT_RING]
    cid = jax.lax.axis_index("c")
    sid = jax.lax.axis_index("s")
    wid = sid * NC + cid
    base = wid * CE_PER_W

    zb = bufs[0]

    @pl.loop(0, CHUNK)
    def _(r):
        @pl.loop(0, D_NODE // 16)
        def _(cc):
            zb[r, pl.ds(cc * 16, 16)] = jnp.zeros((16,), jnp.float32)

    @pl.loop(0, ROWS_PER_SUB // CHUNK)
    def _(t):
        pltpu.sync_copy(zb, acc_sh.at[pl.ds(sid * ROWS_PER_SUB + t * CHUNK, CHUNK)])

    plsc.subcore_barrier()

    for k in range(K_SPLIT):
        mk = msg_arrays[k]
        pltpu.sync_copy(recv4_hbm.at[k, wid], idx2_v)

        def fire_l(j, b, mk=mk):
            src = mk.at[pl.ds(base + j * CHUNK, CHUNK)]
            pltpu.async_copy(src, bufs[b], lsem[b])

        def drain_l(b, mk=mk):
            pltpu.make_async_copy(
                mk.at[pl.ds(0, CHUNK)], bufs[b], lsem[b]).wait()

        def scat(j, b):
            pltpu.sync_copy(bufs[b], acc_sh.at[idx2_v.at[j]], add=True)

        for b in range(SCAT_RING):
            fire_l(b, b)

        @pl.loop(0, (CN_CHUNKS - 7) // SCAT_RING)
        def _(g):
            for b in range(SCAT_RING):
                j = g * SCAT_RING + b
                drain_l(b)
                scat(j, b)
                fire_l(j + SCAT_RING, b)

        for j in range(CN_CHUNKS - 7, CN_CHUNKS):
            b = j % SCAT_RING
            drain_l(b)
            scat(j, b)
            if j + SCAT_RING < CN_CHUNKS:
                fire_l(j + SCAT_RING, b)

    plsc.subcore_barrier()

    rb = sid * ROWS_PER_SUB

    @pl.when(cid == 0)
    def _():
        pltpu.sync_copy(acc_sh.at[pl.ds(rb, ROWS_PER_SUB)], p0_out.at[pl.ds(rb, ROWS_PER_SUB)])

    @pl.when(cid == 1)
    def _():
        pltpu.sync_copy(acc_sh.at[pl.ds(rb, ROWS_PER_SUB)], p1_out.at[pl.ds(rb, ROWS_PER_SUB)])


@jax.jit
def _sc_scatter(m0, m1, m2, m3, m4, receivers):
    part = jax.ShapeDtypeStruct((N_PAD, D_NODE), jnp.float32)
    recv4 = receivers.reshape(K_SPLIT, NW, CN_CHUNKS, CHUNK)
    k = pl.kernel(
        _scatter_body,
        out_type=(part, part),
        mesh=_vector_mesh,
        scratch_types=[
            pltpu.VMEM_SHARED((N_PAD, D_NODE), jnp.float32),
            pltpu.VMEM((CN_CHUNKS, CHUNK), jnp.int32),
            *[pltpu.VMEM((CHUNK, D_NODE), jnp.float32) for _ in range(SCAT_RING)],
            *[pltpu.SemaphoreType.DMA for _ in range(SCAT_RING)],
        ],
    )
    return k(m0, m1, m2, m3, m4, recv4)


E_BLK = 2000


D_EEXT = 32


def _mlp_body(sr_ref, e_ref, w1sr_ref, w1e_ref, w2_ref, b2_ref, o_ref):
    x = sr_ref[...].astype(jnp.bfloat16)
    h = jnp.dot(x, w1sr_ref[...], preferred_element_type=jnp.float32)
    h += jnp.dot(e_ref[...], w1e_ref[...], preferred_element_type=jnp.float32)
    h = jnp.maximum(h, 0.0).astype(jnp.bfloat16)
    o_ref[...] = jnp.dot(h, w2_ref[...], preferred_element_type=jnp.float32) + b2_ref[...]


def _mlp_chunk(k, sr_feat, e_ext, w1sr, w1e_ext, w2, b2r):
    grid = (E_SPLIT // E_BLK,)
    blocks_per_split = E_SPLIT // E_BLK
    in_specs = [
        pl.BlockSpec((E_BLK, 2 * D_NODE), lambda i: (i, 0)),
        pl.BlockSpec((E_BLK, D_EEXT), lambda i: (i + k * blocks_per_split, 0)),
        pl.BlockSpec((2 * D_NODE, D_HID), lambda i: (0, 0)),
        pl.BlockSpec((D_EEXT, D_HID), lambda i: (0, 0)),
        pl.BlockSpec((D_HID, D_OUT), lambda i: (0, 0)),
        pl.BlockSpec((1, D_OUT), lambda i: (0, 0)),
    ]
    out_spec = pl.BlockSpec((E_BLK, D_OUT), lambda i: (i, 0))
    out_shape = jax.ShapeDtypeStruct((E_SPLIT, D_OUT), jnp.float32)
    return pl.pallas_call(
        _mlp_body, grid=grid, in_specs=in_specs,
        out_specs=out_spec, out_shape=out_shape,
    )(sr_feat, e_ext, w1sr, w1e_ext, w2, b2r)


N_BLK = 2000


def _update_body(p0_ref, p1_ref, n_ref, wn_ref, bn_ref, o_ref):
    proj = jnp.dot(n_ref[...].astype(jnp.bfloat16), wn_ref[...],
                   preferred_element_type=jnp.float32)
    o_ref[...] = p0_ref[...] + p1_ref[...] + proj + bn_ref[...]


def _update(p0, p1, nodes, Wn, bn):
    wn = Wn.astype(jnp.bfloat16)
    bnr = bn.reshape(1, D_OUT)
    return pl.pallas_call(
        _update_body,
        grid=(N_NODES // N_BLK,),
        in_specs=[
            pl.BlockSpec((N_BLK, D_OUT), lambda i: (i, 0)),
            pl.BlockSpec((N_BLK, D_OUT), lambda i: (i, 0)),
            pl.BlockSpec((N_BLK, D_NODE), lambda i: (i, 0)),
            pl.BlockSpec((D_NODE, D_OUT), lambda i: (0, 0)),
            pl.BlockSpec((1, D_OUT), lambda i: (0, 0)),
        ],
        out_specs=pl.BlockSpec((N_BLK, D_OUT), lambda i: (i, 0)),
        out_shape=jax.ShapeDtypeStruct((N_NODES, D_OUT), jnp.float32),
    )(p0, p1, nodes, wn, bnr)


def kernel(nodes, edges, senders, receivers, W1, b1, W2, b2, Wn, bn):
    w1sr = W1[:2 * D_NODE].astype(jnp.bfloat16)
    w1e_ext = jnp.concatenate(
        [W1[2 * D_NODE:], b1.reshape(1, D_HID),
         jnp.zeros((D_EEXT - D_EDGE - 1, D_HID), W1.dtype)],
        axis=0).astype(jnp.bfloat16)
    e_ext = jnp.concatenate(
        [edges, jnp.ones((N_EDGES, 1), edges.dtype),
         jnp.zeros((N_EDGES, D_EEXT - D_EDGE - 1), edges.dtype)],
        axis=1).astype(jnp.bfloat16)
    w2 = W2.astype(jnp.bfloat16)
    b2r = b2.reshape(1, D_OUT)
    msgs = []
    for k in range(K_SPLIT):
        sl = slice(k * E_SPLIT, (k + 1) * E_SPLIT)
        sr_feat = _sc_gather(nodes, senders[sl], receivers[sl])
        msgs.append(_mlp_chunk(k, sr_feat, e_ext, w1sr, w1e_ext, w2, b2r))
    p0, p1 = _sc_scatter(*msgs, receivers)
    return _update(p0, p1, nodes, Wn, bn)


# speedup vs baseline: 3.9428x; 1.0394x over previous
import functools

import jax
import jax.numpy as jnp
from jax.experimental import pallas as pl
from jax.experimental.pallas import tpu as pltpu
from jax.experimental.pallas import tpu_sc as plsc

N_NODES = 10000
N_EDGES = 320000
D_NODE = 128
D_EDGE = 16
D_OUT = 128
D_HID = 256

NC = 2
NS = 16
NW = NC * NS

E_PER_W = N_EDGES // NW
CHUNK = 80
N_CHUNKS = E_PER_W // CHUNK

N_PAD = 10240
ROWS_PER_SUB = N_PAD // NS
ZROWS = 128

_vector_mesh = plsc.VectorSubcoreMesh(core_axis_name="c", subcore_axis_name="s")


GRP = 5
K_SPLIT = 5
E_SPLIT = N_EDGES // K_SPLIT
CE_PER_W = E_SPLIT // NW
CN_CHUNKS = CE_PER_W // CHUNK
CN_GRP = CN_CHUNKS // GRP


def _gather_body(nodes_hbm, senders_hbm, receivers_hbm, sr_out,
                 idx_all, *rest):
    bufs = (rest[0:GRP], rest[GRP:2 * GRP])
    gsem = rest[2 * GRP:2 * GRP + 2]
    wsem = rest[2 * GRP + 2:2 * GRP + 4]
    wid = jax.lax.axis_index("s") * NC + jax.lax.axis_index("c")
    base = wid * CE_PER_W

    def phase(src_hbm, col):
        pltpu.sync_copy(src_hbm.at[pl.ds(base, CE_PER_W)], idx_all)

        def fire_g(g, h):
            for b in range(GRP):
                idx = idx_all.at[pl.ds((g * GRP + b) * CHUNK, CHUNK)]
                pltpu.async_copy(nodes_hbm.at[idx], bufs[h][b], gsem[h])

        def drain_g(h):
            for b in range(GRP):
                pltpu.make_async_copy(
                    nodes_hbm.at[pl.ds(0, CHUNK)], bufs[h][b], gsem[h]).wait()

        def fire_w(g, h):
            for b in range(GRP):
                dst = sr_out.at[pl.ds(base + (g * GRP + b) * CHUNK, CHUNK),
                                pl.ds(col, D_NODE)]
                pltpu.async_copy(bufs[h][b], dst, wsem[h])

        def drain_w(h):
            for b in range(GRP):
                pltpu.make_async_copy(
                    bufs[h][b],
                    sr_out.at[pl.ds(base, CHUNK), pl.ds(col, D_NODE)],
                    wsem[h]).wait()

        fire_g(0, 0)
        fire_g(1, 1); drain_g(0); fire_w(0, 0)

        @pl.loop(0, (CN_GRP - 3) // 2)
        def _(u):
            t1 = 2 * u + 1
            drain_w(0); fire_g(t1 + 1, 0); drain_g(1); fire_w(t1, 1)
            t2 = 2 * u + 2
            drain_w(1); fire_g(t2 + 1, 1); drain_g(0); fire_w(t2, 0)

        drain_w(0); fire_g(CN_GRP - 1, 0); drain_g(1); fire_w(CN_GRP - 2, 1)
        drain_w(1); drain_g(0); fire_w(CN_GRP - 1, 0)
        drain_w(0)

    phase(senders_hbm, 0)
    phase(receivers_hbm, D_NODE)


@jax.jit
def _sc_gather(nodes, senders, receivers):
    feat = jax.ShapeDtypeStruct((E_SPLIT, 2 * D_NODE), jnp.float32)
    k = pl.kernel(
        _gather_body,
        out_type=feat,
        mesh=_vector_mesh,
        scratch_types=[
            pltpu.VMEM((CE_PER_W,), jnp.int32),
            *[pltpu.VMEM((CHUNK, D_NODE), jnp.float32) for _ in range(2 * GRP)],
            pltpu.SemaphoreType.DMA,
            pltpu.SemaphoreType.DMA,
            pltpu.SemaphoreType.DMA,
            pltpu.SemaphoreType.DMA,
        ],
    )
    return k(nodes, senders, receivers)


SCAT_RING = 3


def _make_scatter_body(ks):
    def _scatter_body(*refs):
        msg_arrays = refs[:len(ks)]
        recv4_hbm = refs[len(ks)]
        p0_out, p1_out = refs[len(ks) + 1:len(ks) + 3]
        acc_sh, idx2_v = refs[len(ks) + 3:len(ks) + 5]
        rest = refs[len(ks) + 5:]
        _scatter_impl(ks, msg_arrays, recv4_hbm, p0_out, p1_out,
                      acc_sh, idx2_v, rest)
    return _scatter_body


def _scatter_impl(ks, msg_arrays, recv4_hbm, p0_out, p1_out,
                  acc_sh, idx2_v, rest):
    bufs = rest[0:SCAT_RING]
    lsem = rest[SCAT_RING:2 * SCAT_RING]
    cid = jax.lax.axis_index("c")
    sid = jax.lax.axis_index("s")
    wid = sid * NC + cid
    base = wid * CE_PER_W

    zb = bufs[0]

    @pl.loop(0, CHUNK)
    def _(r):
        @pl.loop(0, D_NODE // 16)
        def _(cc):
            zb[r, pl.ds(cc * 16, 16)] = jnp.zeros((16,), jnp.float32)

    @pl.loop(0, ROWS_PER_SUB // CHUNK)
    def _(t):
        pltpu.sync_copy(zb, acc_sh.at[pl.ds(sid * ROWS_PER_SUB + t * CHUNK, CHUNK)])

    plsc.subcore_barrier()

    for seg, k in enumerate(ks):
        mk = msg_arrays[seg]
        pltpu.sync_copy(recv4_hbm.at[k, wid], idx2_v)

        def fire_l(j, b, mk=mk):
            src = mk.at[pl.ds(base + j * CHUNK, CHUNK)]
            pltpu.async_copy(src, bufs[b], lsem[b])

        def drain_l(b, mk=mk):
            pltpu.make_async_copy(
                mk.at[pl.ds(0, CHUNK)], bufs[b], lsem[b]).wait()

        def scat(j, b):
            pltpu.sync_copy(bufs[b], acc_sh.at[idx2_v.at[j]], add=True)

        for b in range(SCAT_RING):
            fire_l(b, b)

        @pl.loop(0, (CN_CHUNKS - 7) // SCAT_RING)
        def _(g):
            for b in range(SCAT_RING):
                j = g * SCAT_RING + b
                drain_l(b)
                scat(j, b)
                fire_l(j + SCAT_RING, b)

        for j in range(CN_CHUNKS - 7, CN_CHUNKS):
            b = j % SCAT_RING
            drain_l(b)
            scat(j, b)
            if j + SCAT_RING < CN_CHUNKS:
                fire_l(j + SCAT_RING, b)

    plsc.subcore_barrier()

    rb = sid * ROWS_PER_SUB

    @pl.when(cid == 0)
    def _():
        pltpu.sync_copy(acc_sh.at[pl.ds(rb, ROWS_PER_SUB)], p0_out.at[pl.ds(rb, ROWS_PER_SUB)])

    @pl.when(cid == 1)
    def _():
        pltpu.sync_copy(acc_sh.at[pl.ds(rb, ROWS_PER_SUB)], p1_out.at[pl.ds(rb, ROWS_PER_SUB)])


def _sc_scatter(msgs, receivers, ks):
    part = jax.ShapeDtypeStruct((N_PAD, D_NODE), jnp.float32)
    recv4 = receivers.reshape(K_SPLIT, NW, CN_CHUNKS, CHUNK)
    k = pl.kernel(
        _make_scatter_body(ks),
        out_type=(part, part),
        mesh=_vector_mesh,
        scratch_types=[
            pltpu.VMEM_SHARED((N_PAD, D_NODE), jnp.float32),
            pltpu.VMEM((CN_CHUNKS, CHUNK), jnp.int32),
            *[pltpu.VMEM((CHUNK, D_NODE), jnp.float32) for _ in range(SCAT_RING)],
            *[pltpu.SemaphoreType.DMA for _ in range(SCAT_RING)],
        ],
    )
    return k(*msgs, recv4)


E_BLK = 2000


D_EEXT = 32


def _mlp_body(sr_ref, e_ref, w1sr_ref, w1e_ref, w2_ref, b2_ref, o_ref):
    x = sr_ref[...].astype(jnp.bfloat16)
    h = jnp.dot(x, w1sr_ref[...], preferred_element_type=jnp.float32)
    h += jnp.dot(e_ref[...], w1e_ref[...], preferred_element_type=jnp.float32)
    h = jnp.maximum(h, 0.0).astype(jnp.bfloat16)
    o_ref[...] = jnp.dot(h, w2_ref[...], preferred_element_type=jnp.float32) + b2_ref[...]


def _mlp_chunk(k, sr_feat, e_ext, w1sr, w1e_ext, w2, b2r):
    grid = (E_SPLIT // E_BLK,)
    blocks_per_split = E_SPLIT // E_BLK
    in_specs = [
        pl.BlockSpec((E_BLK, 2 * D_NODE), lambda i: (i, 0)),
        pl.BlockSpec((E_BLK, D_EEXT), lambda i: (i + k * blocks_per_split, 0)),
        pl.BlockSpec((2 * D_NODE, D_HID), lambda i: (0, 0)),
        pl.BlockSpec((D_EEXT, D_HID), lambda i: (0, 0)),
        pl.BlockSpec((D_HID, D_OUT), lambda i: (0, 0)),
        pl.BlockSpec((1, D_OUT), lambda i: (0, 0)),
    ]
    out_spec = pl.BlockSpec((E_BLK, D_OUT), lambda i: (i, 0))
    out_shape = jax.ShapeDtypeStruct((E_SPLIT, D_OUT), jnp.float32)
    return pl.pallas_call(
        _mlp_body, grid=grid, in_specs=in_specs,
        out_specs=out_spec, out_shape=out_shape,
    )(sr_feat, e_ext, w1sr, w1e_ext, w2, b2r)


N_BLK = 2000


def _update_body(p0_ref, p1_ref, p2_ref, p3_ref, n_ref, wn_ref, bn_ref, o_ref):
    proj = jnp.dot(n_ref[...].astype(jnp.bfloat16), wn_ref[...],
                   preferred_element_type=jnp.float32)
    o_ref[...] = ((p0_ref[...] + p1_ref[...]) + (p2_ref[...] + p3_ref[...])
                  + proj + bn_ref[...])


def _update(parts, nodes, Wn, bn):
    wn = Wn.astype(jnp.bfloat16)
    bnr = bn.reshape(1, D_OUT)
    return pl.pallas_call(
        _update_body,
        grid=(N_NODES // N_BLK,),
        in_specs=[
            pl.BlockSpec((N_BLK, D_OUT), lambda i: (i, 0)),
            pl.BlockSpec((N_BLK, D_OUT), lambda i: (i, 0)),
            pl.BlockSpec((N_BLK, D_OUT), lambda i: (i, 0)),
            pl.BlockSpec((N_BLK, D_OUT), lambda i: (i, 0)),
            pl.BlockSpec((N_BLK, D_NODE), lambda i: (i, 0)),
            pl.BlockSpec((D_NODE, D_OUT), lambda i: (0, 0)),
            pl.BlockSpec((1, D_OUT), lambda i: (0, 0)),
        ],
        out_specs=pl.BlockSpec((N_BLK, D_OUT), lambda i: (i, 0)),
        out_shape=jax.ShapeDtypeStruct((N_NODES, D_OUT), jnp.float32),
    )(*parts, nodes, wn, bnr)


def kernel(nodes, edges, senders, receivers, W1, b1, W2, b2, Wn, bn):
    w1sr = W1[:2 * D_NODE].astype(jnp.bfloat16)
    w1e_ext = jnp.concatenate(
        [W1[2 * D_NODE:], b1.reshape(1, D_HID),
         jnp.zeros((D_EEXT - D_EDGE - 1, D_HID), W1.dtype)],
        axis=0).astype(jnp.bfloat16)
    e_ext = jnp.concatenate(
        [edges, jnp.ones((N_EDGES, 1), edges.dtype),
         jnp.zeros((N_EDGES, D_EEXT - D_EDGE - 1), edges.dtype)],
        axis=1).astype(jnp.bfloat16)
    w2 = W2.astype(jnp.bfloat16)
    b2r = b2.reshape(1, D_OUT)
    msgs = []
    for k in range(K_SPLIT):
        sl = slice(k * E_SPLIT, (k + 1) * E_SPLIT)
        sr_feat = _sc_gather(nodes, senders[sl], receivers[sl])
        msgs.append(_mlp_chunk(k, sr_feat, e_ext, w1sr, w1e_ext, w2, b2r))
    p0, p1 = _sc_scatter(msgs[:4], receivers, ks=(0, 1, 2, 3))
    p2, p3 = _sc_scatter(msgs[4:], receivers, ks=(4,))
    return _update((p0, p1, p2, p3), nodes, Wn, bn)
